# Initial kernel scaffold; baseline (speedup 1.0000x reference)
import jax
import jax.numpy as jnp
from jax.experimental import pallas as pl


def kernel(Y, P, Lam, M, Kset):
    raise NotImplementedError("write your pallas kernel here")


# speedup vs baseline: 1.0748x; 1.0748x over previous
import jax
import jax.numpy as jnp
from jax.experimental import pallas as pl
from jax.experimental.pallas import tpu as pltpu
from jax.experimental.pallas import tpu_sc as plsc

RHO = 1.0
BETA = 0.5
TAU = 0.1
EPS = 1e-9

_N_TOK = 2048
_K = 32
_D = 768
_B = 8

_GATHER_WINDOW = 128
_ROW_SPLIT = 2


def _sc_gather(M, idx_flat):
    num_idx = idx_flat.shape[0]
    d = M.shape[1]
    ds = d // _ROW_SPLIT
    M2 = M.reshape(M.shape[0] * _ROW_SPLIT, ds)
    idx2 = (idx_flat[:, None] * _ROW_SPLIT
            + jnp.arange(_ROW_SPLIT, dtype=idx_flat.dtype)[None, :])
    num2 = num_idx * _ROW_SPLIT
    indices = idx2.reshape(num2 // _GATHER_WINDOW, _GATHER_WINDOW)
    mesh = plsc.VectorSubcoreMesh(core_axis_name="core",
                                  subcore_axis_name="subcore")

    @pl.kernel(out_type=jax.ShapeDtypeStruct((num2, ds), M.dtype),
               mesh=mesh)
    def gather_kernel(m_hbm, i_hbm, o_hbm):
        def body(i_vmem, o_vmem):
            pltpu.sync_copy(m_hbm.at[i_vmem.at[0]], o_vmem)

        pltpu.emit_pipeline(
            body,
            grid=(num2 // _GATHER_WINDOW,),
            in_specs=[pl.BlockSpec((1, _GATHER_WINDOW), lambda i: (i, 0))],
            out_specs=[pl.BlockSpec((_GATHER_WINDOW, ds), lambda i: (i, 0))],
            core_axis_name=("core", "subcore"),
            dimension_semantics=(pltpu.PARALLEL,),
        )(i_hbm, o_hbm)

    return gather_kernel(M2, indices).reshape(num_idx, d)


def _tc_body(y_ref, p_ref, lam_ref, t_ref, pnew_ref, lamnew_ref):
    T3 = t_ref[...]
    P3 = p_ref[...]
    Y = y_ref[...]
    Lam = lam_ref[...]

    S = jnp.sum(P3, axis=0, keepdims=True)
    Pn3 = P3 / (S + EPS)
    Tb = T3.astype(jnp.bfloat16).astype(jnp.float32)
    Pnb = Pn3.astype(jnp.bfloat16).astype(jnp.float32)
    yfp = jnp.sum(Tb * Pnb, axis=0)
    Xi = Lam + RHO * (Y - yfp)
    Xib = Xi.astype(jnp.bfloat16).astype(jnp.float32)
    scores = jnp.sum(Tb * Xib[None, :, :], axis=2,
                     keepdims=True)
    yp = jnp.sum(Tb * Pn3, axis=0)
    s2 = jnp.sum(Tb * yp[None, :, :], axis=2, keepdims=True)
    sq = jnp.sum(T3 * T3, axis=2, keepdims=True)
    logits = (jnp.log(Pn3 + EPS) - BETA * scores
              - TAU * sq + (2.0 * TAU) * s2)
    m = jnp.max(logits, axis=0, keepdims=True)
    e = jnp.exp(logits - m)
    Pnew3 = e / jnp.sum(e, axis=0, keepdims=True)
    Pnewb = Pnew3.astype(jnp.bfloat16).astype(jnp.float32)
    yfp2 = jnp.sum(Tb * Pnewb, axis=0)
    pnew_ref[...] = Pnew3
    lamnew_ref[...] = Lam + RHO * (Y - yfp2)


def _tc_compute(Y, P, Lam, T):
    n, d = Y.shape
    k = P.shape[1]
    grid = (n // _B,)
    pnew_t, lam_new = pl.pallas_call(
        _tc_body,
        grid=grid,
        in_specs=[
            pl.BlockSpec((_B, d), lambda i: (i, 0)),
            pl.BlockSpec((k, _B, 1), lambda i: (0, i, 0)),
            pl.BlockSpec((_B, d), lambda i: (i, 0)),
            pl.BlockSpec((k, _B, d), lambda i: (0, i, 0)),
        ],
        out_specs=[
            pl.BlockSpec((k, _B, 1), lambda i: (0, i, 0)),
            pl.BlockSpec((_B, d), lambda i: (i, 0)),
        ],
        out_shape=[
            jax.ShapeDtypeStruct((k, n, 1), jnp.float32),
            jax.ShapeDtypeStruct((n, d), jnp.float32),
        ],
    )(Y, P.T.reshape(k, n, 1), Lam, T)
    return pnew_t.reshape(k, n).T, lam_new


def kernel(Y, P, Lam, M, Kset):
    n, k = Kset.shape
    T = _sc_gather(M, Kset.T.reshape(n * k))
    P_new, Lam_new = _tc_compute(Y, P, Lam, T.reshape(k, n, _D))
    return (P_new, Lam_new)
